# Initial kernel scaffold; baseline (speedup 1.0000x reference)
import functools

import jax
import jax.numpy as jnp
import numpy as np
from jax import lax
from jax.experimental import pallas as pl
from jax.experimental.pallas import tpu as pltpu
from jax.experimental.pallas import tpu_sc as plsc

NLAT, NLON, KERNEL = 180, 360, 30
C = 64
L = 16
NG = C // L
NC, NS = 2, 16
NW = NC * NS


def _grid_points(H, W):
    lat = np.pi * (np.arange(H) + 0.5) / H - np.pi / 2.0
    lon = 2.0 * np.pi * np.arange(W) / W
    la, lo = np.meshgrid(lat, lon, indexing='ij')
    pts = np.stack([np.cos(la) * np.cos(lo), np.cos(la) * np.sin(lo),
                    np.sin(la)], axis=-1)
    return pts.reshape(H * W, 3)


def _compressed_weights(in_ratio, out_ratio):
    Hi, Wi = int(round(NLAT * in_ratio)), int(round(NLON * in_ratio))
    Ho, Wo = int(round(NLAT * out_ratio)), int(round(NLON * out_ratio))
    P = _grid_points(Hi, Wi)
    M = _grid_points(Ho, Wo).reshape(Ho, Wo, 3)[:, :2]
    ci = np.clip(((np.arange(Ho) + 0.5) * Hi / Ho).astype(np.int64), 0, Hi - 1)
    cj = (((np.arange(2) + 0.5) * Wi / Wo).astype(np.int64)) % Wi
    di = np.array([-2, -1, 0, 1, 2], dtype=np.int64)
    dj = np.array([-3, -2, -1, 0, 1, 2], dtype=np.int64)
    ii = np.clip(ci[:, None, None, None] + di[None, None, :, None], 0, Hi - 1)
    jj = (cj[None, :, None, None] + dj[None, None, None, :]) % Wi
    idx = (ii * Wi + jj).reshape(Ho, 2, KERNEL)
    dots = np.einsum('hpd,hpkd->hpk', M, P[idx])
    kappa = (Hi * Hi) / 4.0
    w = np.exp(kappa * (dots - 1.0))
    w = w / w.sum(axis=2, keepdims=True)
    return w.astype(np.float32)


def _make_layer(in_ratio, out_ratio, max_pairs, tiles_hi):
    Hi, Wi = int(round(NLAT * in_ratio)), int(round(NLON * in_ratio))
    Ho, Wo = int(round(NLAT * out_ratio)), int(round(NLON * out_ratio))
    assert tiles_hi * max_pairs + (NW - tiles_hi) * (max_pairs - 1) == Ho // 2
    n_slots = max_pairs + 4
    BW = Wi + 5
    RW = BW * C
    wrow = 2 * KERNEL * L

    wc = _compressed_weights(in_ratio, out_ratio)
    wsp = np.repeat(wc.reshape(Ho, 2 * KERNEL), L, axis=1)
    wsp = jnp.asarray(wsp.reshape(-1))

    mesh = plsc.VectorSubcoreMesh(core_axis_name="c", subcore_axis_name="s",
                                  num_cores=NC, num_subcores=NS)

    @functools.partial(
        pl.kernel,
        out_type=jax.ShapeDtypeStruct((Ho * Wo * C,), jnp.float32),
        mesh=mesh,
        scratch_types=[
            pltpu.VMEM((n_slots * RW,), jnp.float32),
            pltpu.VMEM((Wo * C,), jnp.float32),
            pltpu.VMEM((wrow,), jnp.float32),
            pltpu.VMEM((C,), jnp.float32),
        ],
    )
    def layer(x_hbm, w_hbm, b_hbm, out_hbm, inbuf, outbuf, wbuf, bbuf):
        wid = lax.axis_index("s") * NC + lax.axis_index("c")
        npairs = jnp.where(wid < tiles_hi, max_pairs, max_pairs - 1)
        u0 = jnp.where(wid < tiles_hi, max_pairs * wid,
                       max_pairs * tiles_hi + (max_pairs - 1) * (wid - tiles_hi))

        pltpu.sync_copy(b_hbm, bbuf)
        for r in range(n_slots):
            i_r = jnp.clip(u0 - 2 + r, 0, Hi - 1)
            src = i_r * (Wi * C)
            dst = r * RW
            pltpu.sync_copy(x_hbm.at[pl.ds(src, Wi * C)],
                            inbuf.at[pl.ds(dst + 3 * C, Wi * C)])
            pltpu.sync_copy(x_hbm.at[pl.ds(src + (Wi - 3) * C, 3 * C)],
                            inbuf.at[pl.ds(dst, 3 * C)])
            pltpu.sync_copy(x_hbm.at[pl.ds(src, 2 * C)],
                            inbuf.at[pl.ds(dst + (3 + Wi) * C, 2 * C)])

        bv = [bbuf[pl.ds(L * g, L)] for g in range(NG)]

        for rr in range(2 * max_pairs):
            @pl.when(rr < 2 * npairs)
            def _():
                ho = 2 * u0 + rr
                pltpu.sync_copy(w_hbm.at[pl.ds(ho * wrow, wrow)], wbuf)

                def body(jp, carry):
                    col = jp * C
                    acc0 = list(bv)
                    acc1 = list(bv)
                    for a in range(5):
                        base = (rr // 2 + a) * RW + col
                        for b in range(6):
                            k = a * 6 + b
                            w0 = wbuf[pl.ds(k * L, L)]
                            w1 = wbuf[pl.ds((KERNEL + k) * L, L)]
                            off = base + b * C
                            for g in range(NG):
                                v = inbuf[pl.ds(off + L * g, L)]
                                acc0[g] = acc0[g] + v * w0
                                acc1[g] = acc1[g] + v * w1
                    for g in range(NG):
                        outbuf[pl.ds(2 * col + L * g, L)] = acc0[g]
                        outbuf[pl.ds(2 * col + C + L * g, L)] = acc1[g]
                    return carry

                lax.fori_loop(0, Wo // 2, body, 0)
                pltpu.sync_copy(outbuf, out_hbm.at[pl.ds(ho * (Wo * C), Wo * C)])

    def apply(x, bias):
        return layer(x, wsp, bias)

    return apply


_layer1 = _make_layer(0.25, 0.5, 2, 13)
_layer2 = _make_layer(0.5, 1.0, 3, 26)


@jax.jit
def kernel(x, b1, b2):
    xf = x.reshape(-1)
    h = _layer1(xf, b1)
    y = _layer2(h, b2)
    return y.reshape(1, NLAT * NLON, C)


# speedup vs baseline: 19.2145x; 19.2145x over previous
import functools

import jax
import jax.numpy as jnp
import numpy as np
from jax import lax
from jax.experimental import pallas as pl
from jax.experimental.pallas import tpu as pltpu
from jax.experimental.pallas import tpu_sc as plsc

NLAT, NLON, KERNEL = 180, 360, 30
C = 64
L = 16
NG = C // L
NC, NS = 2, 16
NW = NC * NS


def _grid_points(H, W):
    lat = np.pi * (np.arange(H) + 0.5) / H - np.pi / 2.0
    lon = 2.0 * np.pi * np.arange(W) / W
    la, lo = np.meshgrid(lat, lon, indexing='ij')
    pts = np.stack([np.cos(la) * np.cos(lo), np.cos(la) * np.sin(lo),
                    np.sin(la)], axis=-1)
    return pts.reshape(H * W, 3)


def _compressed_weights(in_ratio, out_ratio):
    Hi, Wi = int(round(NLAT * in_ratio)), int(round(NLON * in_ratio))
    Ho, Wo = int(round(NLAT * out_ratio)), int(round(NLON * out_ratio))
    P = _grid_points(Hi, Wi)
    M = _grid_points(Ho, Wo).reshape(Ho, Wo, 3)[:, :2]
    ci = np.clip(((np.arange(Ho) + 0.5) * Hi / Ho).astype(np.int64), 0, Hi - 1)
    cj = (((np.arange(2) + 0.5) * Wi / Wo).astype(np.int64)) % Wi
    di = np.array([-2, -1, 0, 1, 2], dtype=np.int64)
    dj = np.array([-3, -2, -1, 0, 1, 2], dtype=np.int64)
    ii = np.clip(ci[:, None, None, None] + di[None, None, :, None], 0, Hi - 1)
    jj = (cj[None, :, None, None] + dj[None, None, None, :]) % Wi
    idx = (ii * Wi + jj).reshape(Ho, 2, KERNEL)
    dots = np.einsum('hpd,hpkd->hpk', M, P[idx])
    kappa = (Hi * Hi) / 4.0
    w = np.exp(kappa * (dots - 1.0))
    w = w / w.sum(axis=2, keepdims=True)
    return w.astype(np.float32)


def _make_layer(in_ratio, out_ratio, max_pairs, tiles_hi):
    Hi, Wi = int(round(NLAT * in_ratio)), int(round(NLON * in_ratio))
    Ho, Wo = int(round(NLAT * out_ratio)), int(round(NLON * out_ratio))
    assert tiles_hi * max_pairs + (NW - tiles_hi) * (max_pairs - 1) == Ho // 2
    n_slots = max_pairs + 4
    BW = Wi + 5
    RW = BW * C
    wrow = 2 * KERNEL * L

    wc = _compressed_weights(in_ratio, out_ratio)
    wsp = np.repeat(wc.reshape(Ho, 2 * KERNEL), L, axis=1)
    wsp = wsp.reshape(-1)

    def layer_body(x_hbm, w_hbm, b_hbm, out_hbm, inbuf, outbuf, wbuf, bbuf):
        wid = lax.axis_index("s") * NC + lax.axis_index("c")
        npairs = jnp.where(wid < tiles_hi, max_pairs, max_pairs - 1)
        u0 = jnp.where(wid < tiles_hi, max_pairs * wid,
                       max_pairs * tiles_hi + (max_pairs - 1) * (wid - tiles_hi))

        pltpu.sync_copy(b_hbm, bbuf)
        for r in range(n_slots):
            i_r = jnp.clip(u0 - 2 + r, 0, Hi - 1)
            src = i_r * (Wi * C)
            dst = r * RW
            pltpu.sync_copy(x_hbm.at[pl.ds(src, Wi * C)],
                            inbuf.at[pl.ds(dst + 3 * C, Wi * C)])
            pltpu.sync_copy(x_hbm.at[pl.ds(src + (Wi - 3) * C, 3 * C)],
                            inbuf.at[pl.ds(dst, 3 * C)])
            pltpu.sync_copy(x_hbm.at[pl.ds(src, 2 * C)],
                            inbuf.at[pl.ds(dst + (3 + Wi) * C, 2 * C)])

        bv = [bbuf[pl.ds(L * g, L)] for g in range(NG)]

        for rr in range(2 * max_pairs):
            @pl.when(rr < 2 * npairs)
            def _():
                ho = 2 * u0 + rr
                pltpu.sync_copy(w_hbm.at[pl.ds(ho * wrow, wrow)], wbuf)

                def body(jp, carry):
                    col = jp * C
                    acc0 = list(bv)
                    acc1 = list(bv)
                    for a in range(5):
                        base = (rr // 2 + a) * RW + col
                        for b in range(6):
                            k = a * 6 + b
                            w0 = wbuf[pl.ds(k * L, L)]
                            w1 = wbuf[pl.ds((KERNEL + k) * L, L)]
                            off = base + b * C
                            for g in range(NG):
                                v = inbuf[pl.ds(off + L * g, L)]
                                acc0[g] = acc0[g] + v * w0
                                acc1[g] = acc1[g] + v * w1
                    for g in range(NG):
                        outbuf[pl.ds(2 * col + L * g, L)] = acc0[g]
                        outbuf[pl.ds(2 * col + C + L * g, L)] = acc1[g]
                    return carry

                lax.fori_loop(0, Wo // 2, body, 0)
                pltpu.sync_copy(outbuf, out_hbm.at[pl.ds(ho * (Wo * C), Wo * C)])

    built = []

    def apply(x, bias):
        if not built:
            mesh = plsc.VectorSubcoreMesh(
                core_axis_name="c", subcore_axis_name="s",
                num_cores=NC, num_subcores=NS)
            built.append(pl.kernel(
                layer_body,
                out_type=jax.ShapeDtypeStruct((Ho * Wo * C,), jnp.float32),
                mesh=mesh,
                scratch_types=[
                    pltpu.VMEM((n_slots * RW,), jnp.float32),
                    pltpu.VMEM((Wo * C,), jnp.float32),
                    pltpu.VMEM((wrow,), jnp.float32),
                    pltpu.VMEM((C,), jnp.float32),
                ],
            ))
        return built[0](x, jnp.asarray(wsp), bias)

    return apply


_layer1 = _make_layer(0.25, 0.5, 2, 13)
_layer2 = _make_layer(0.5, 1.0, 3, 26)


@jax.jit
def kernel(x, b1, b2):
    xf = x.reshape(-1)
    h = _layer1(xf, b1)
    y = _layer2(h, b2)
    return y.reshape(1, NLAT * NLON, C)
